# baseline (speedup 1.0000x reference)
import numpy as np

import jax
import jax.numpy as jnp
from jax.experimental import pallas as pl
from jax.experimental.pallas import tpu as pltpu

_BN = 2048


def _tap_select(cin):
    sel = np.zeros((288, cin, 6, 6), np.float32)
    for a in range(6):
        for d in range(6):
            base = ((a // 2) * 3 + (d // 2)) * 4 + 2 * (a % 2) + d % 2
            for c in range(cin):
                sel[base * 8 + c, c, a, d] = 1.0
    return sel


def _col_select(win_w, kpad, k_max):
    w = np.arange(win_w)[:, None, None]
    d = np.arange(6)[None, :, None]
    k = np.arange(kpad)[None, None, :]
    return ((w == 2 * k + d) & (k <= k_max)).astype(np.float32)


def _toeplitz(w_packed, cin, win_w, kpad, k_max):
    cout = w_packed.shape[1]
    t = jnp.einsum('qot,tcad,wdk->qokacw', w_packed,
                   _tap_select(cin), _col_select(win_w, kpad, k_max))
    return t.reshape(4 * cout * kpad, 6 * cin * win_w)


def _fused_kernel(x_ref, t1_ref, b1_ref, t2_ref, b2_ref,
                  f1_ref, c1_ref, f2_ref, c2_ref, f3_ref, c3_ref, o_ref):
    bn = o_ref.shape[1]
    X = x_ref[...].reshape(32, 96, bn)

    t1 = t1_ref[...]
    rows = []
    for h in range(14):
        win = X[2 * h:2 * h + 6].reshape(576, bn)
        z = jnp.dot(t1, win, preferred_element_type=jnp.float32)
        rows.append(jnp.max(z.reshape(4, 96, bn), axis=0))
    y = jnp.stack(rows, axis=0)
    p1 = jnp.maximum(y + b1_ref[...], 0.0).astype(jnp.bfloat16)

    t2 = t2_ref[...]
    rows2 = []
    for h in range(5):
        win = p1[2 * h:2 * h + 6].reshape(576, bn)
        z = jnp.dot(t2, win, preferred_element_type=jnp.float32)
        rows2.append(jnp.max(z.reshape(4, 128, bn), axis=0))
    y2 = jnp.stack(rows2, axis=0)
    p2 = jnp.maximum(y2 + b2_ref[...], 0.0).astype(jnp.bfloat16)
    flat = p2.reshape(640, bn)

    h1 = jnp.dot(f1_ref[...], flat, preferred_element_type=jnp.float32)
    h1 = jnp.maximum(h1 + c1_ref[...], 0.0).astype(jnp.bfloat16)
    h2 = jnp.dot(f2_ref[...], h1, preferred_element_type=jnp.float32)
    h2 = jnp.maximum(h2 + c2_ref[...], 0.0).astype(jnp.bfloat16)
    z3 = jnp.dot(f3_ref[...], h2, preferred_element_type=jnp.float32)
    z3 = z3 + c3_ref[...]
    m = jnp.max(z3, axis=0, keepdims=True)
    e = jnp.exp(z3 - m)
    s = jnp.sum(e, axis=0, keepdims=True)
    o_ref[...] = z3 - m - jnp.log(s)


def kernel(x, w1, b1, w2, b2, wf1, bf1, wf2, bf2, wf3, bf3):
    n = x.shape[0]
    n_pad = ((n + _BN - 1) // _BN) * _BN
    if n_pad != n:
        x = jnp.pad(x, ((0, n_pad - n), (0, 0), (0, 0), (0, 0)))

    xt = jnp.transpose(x, (2, 1, 3, 0)).astype(jnp.bfloat16)

    t1 = _toeplitz(w1, 3, 32, 16, 13).astype(jnp.bfloat16)
    t2 = _toeplitz(w2, 6, 16, 8, 4).astype(jnp.bfloat16)
    b1s = (jnp.tile(b1, (1, 16)) *
           (np.arange(16) < 14).astype(np.float32)).reshape(96, 1)
    b2s = (jnp.tile(b2, (1, 8)) *
           (np.arange(8) < 5).astype(np.float32)).reshape(128, 1)
    f1 = jnp.pad(wf1.reshape(16, 5, 5, 120),
                 ((0, 0), (0, 0), (0, 3), (0, 0)))
    f1 = jnp.transpose(f1, (1, 0, 2, 3)).reshape(640, 120).T.astype(jnp.bfloat16)
    f2 = wf2.T.astype(jnp.bfloat16)
    f3 = wf3.T.astype(jnp.bfloat16)
    c1 = bf1.reshape(120, 1)
    c2 = bf2.reshape(84, 1)
    c3 = bf3.reshape(10, 1)

    def whole(shape):
        nd = len(shape)
        return pl.BlockSpec(shape, lambda i, nd=nd: (0,) * nd)

    out = pl.pallas_call(
        _fused_kernel,
        out_shape=jax.ShapeDtypeStruct((10, n_pad), jnp.float32),
        grid=(n_pad // _BN,),
        in_specs=[pl.BlockSpec((32, 3, 32, _BN), lambda i: (0, 0, 0, i)),
                  whole(t1.shape), whole(b1s.shape),
                  whole(t2.shape), whole(b2s.shape),
                  whole(f1.shape), whole(c1.shape),
                  whole(f2.shape), whole(c2.shape),
                  whole(f3.shape), whole(c3.shape)],
        out_specs=pl.BlockSpec((10, _BN), lambda i: (0, i)),
        compiler_params=pltpu.CompilerParams(
            dimension_semantics=("parallel",)),
    )(xt, t1, b1s, t2, b2s, f1, c1, f2, c2, f3, c3)
    return out.T[:n]


# speedup vs baseline: 1.0052x; 1.0052x over previous
import numpy as np

import jax
import jax.numpy as jnp
from jax.experimental import pallas as pl
from jax.experimental.pallas import tpu as pltpu

_BN = 1024


def _tap_select(cin):
    sel = np.zeros((288, cin, 6, 6), np.float32)
    for a in range(6):
        for d in range(6):
            base = ((a // 2) * 3 + (d // 2)) * 4 + 2 * (a % 2) + d % 2
            for c in range(cin):
                sel[base * 8 + c, c, a, d] = 1.0
    return sel


def _col_select(win_w, kpad, k_max):
    w = np.arange(win_w)[:, None, None]
    d = np.arange(6)[None, :, None]
    k = np.arange(kpad)[None, None, :]
    return ((w == 2 * k + d) & (k <= k_max)).astype(np.float32)


def _toeplitz(w_packed, cin, win_w, kpad, k_max):
    cout = w_packed.shape[1]
    t = jnp.einsum('qot,tcad,wdk->qokacw', w_packed,
                   _tap_select(cin), _col_select(win_w, kpad, k_max))
    return t.reshape(4 * cout * kpad, 6 * cin * win_w)


def _fused_kernel(x_ref, t1_ref, b1_ref, t2_ref, b2_ref,
                  f1_ref, c1_ref, f2_ref, c2_ref, f3_ref, c3_ref, o_ref):
    bn = o_ref.shape[1]
    X = x_ref[...].reshape(32, 96, bn)

    t1 = t1_ref[...]
    rows = []
    for h in range(14):
        win = X[2 * h:2 * h + 6].reshape(576, bn)
        z = jnp.dot(t1, win, preferred_element_type=jnp.float32)
        rows.append(jnp.max(z.reshape(4, 96, bn), axis=0))
    y = jnp.stack(rows, axis=0)
    p1 = jnp.maximum(y + b1_ref[...], 0.0).astype(jnp.bfloat16)

    t2 = t2_ref[...]
    rows2 = []
    for h in range(5):
        win = p1[2 * h:2 * h + 6].reshape(576, bn)
        z = jnp.dot(t2, win, preferred_element_type=jnp.float32)
        rows2.append(jnp.max(z.reshape(4, 128, bn), axis=0))
    y2 = jnp.stack(rows2, axis=0)
    p2 = jnp.maximum(y2 + b2_ref[...], 0.0).astype(jnp.bfloat16)
    flat = p2.reshape(640, bn)

    h1 = jnp.dot(f1_ref[...], flat, preferred_element_type=jnp.float32)
    h1 = jnp.maximum(h1 + c1_ref[...], 0.0).astype(jnp.bfloat16)
    h2 = jnp.dot(f2_ref[...], h1, preferred_element_type=jnp.float32)
    h2 = jnp.maximum(h2 + c2_ref[...], 0.0).astype(jnp.bfloat16)
    z3 = jnp.dot(f3_ref[...], h2, preferred_element_type=jnp.float32)
    z3 = z3 + c3_ref[...]
    m = jnp.max(z3, axis=0, keepdims=True)
    e = jnp.exp(z3 - m)
    s = jnp.sum(e, axis=0, keepdims=True)
    o_ref[...] = z3 - m - jnp.log(s)


def kernel(x, w1, b1, w2, b2, wf1, bf1, wf2, bf2, wf3, bf3):
    n = x.shape[0]
    n_pad = ((n + _BN - 1) // _BN) * _BN
    if n_pad != n:
        x = jnp.pad(x, ((0, n_pad - n), (0, 0), (0, 0), (0, 0)))

    xt = jnp.transpose(x, (2, 1, 3, 0)).astype(jnp.bfloat16)

    t1 = _toeplitz(w1, 3, 32, 16, 13).astype(jnp.bfloat16)
    t2 = _toeplitz(w2, 6, 16, 8, 4).astype(jnp.bfloat16)
    b1s = (jnp.tile(b1, (1, 16)) *
           (np.arange(16) < 14).astype(np.float32)).reshape(96, 1)
    b2s = (jnp.tile(b2, (1, 8)) *
           (np.arange(8) < 5).astype(np.float32)).reshape(128, 1)
    f1 = jnp.pad(wf1.reshape(16, 5, 5, 120),
                 ((0, 0), (0, 0), (0, 3), (0, 0)))
    f1 = jnp.transpose(f1, (1, 0, 2, 3)).reshape(640, 120).T.astype(jnp.bfloat16)
    f2 = wf2.T.astype(jnp.bfloat16)
    f3 = wf3.T.astype(jnp.bfloat16)
    c1 = bf1.reshape(120, 1)
    c2 = bf2.reshape(84, 1)
    c3 = bf3.reshape(10, 1)

    def whole(shape):
        nd = len(shape)
        return pl.BlockSpec(shape, lambda i, nd=nd: (0,) * nd)

    out = pl.pallas_call(
        _fused_kernel,
        out_shape=jax.ShapeDtypeStruct((10, n_pad), jnp.float32),
        grid=(n_pad // _BN,),
        in_specs=[pl.BlockSpec((32, 3, 32, _BN), lambda i: (0, 0, 0, i)),
                  whole(t1.shape), whole(b1s.shape),
                  whole(t2.shape), whole(b2s.shape),
                  whole(f1.shape), whole(c1.shape),
                  whole(f2.shape), whole(c2.shape),
                  whole(f3.shape), whole(c3.shape)],
        out_specs=pl.BlockSpec((10, _BN), lambda i: (0, i)),
        compiler_params=pltpu.CompilerParams(
            dimension_semantics=("parallel",)),
    )(xt, t1, b1s, t2, b2s, f1, c1, f2, c2, f3, c3)
    return out.T[:n]
